# baseline (speedup 1.0000x reference)
import functools
import jax
import jax.numpy as jnp
import numpy as np
from jax.experimental import pallas as pl
from jax.experimental.pallas import tpu as pltpu

_EPS = 1e-6
_K = 20
_KPAD = 32
_RB = 256


def _topk_body(xfb_ref, xfa_ref, tagb_ref, taga_ref, idx_ref):
    xb = xfb_ref[0]
    xa = xfa_ref[0]
    dot = jax.lax.dot_general(xb, xa, (((0,), (0,)), ((), ())),
                              preferred_element_type=jnp.float32)
    sqb = jnp.sum(xb * xb, axis=0)
    sqa = jnp.sum(xa * xa, axis=0)
    pd = 2.0 * dot - sqb[:, None] - sqa[None, :]
    same = tagb_ref[0, 0][:, None] == taga_ref[0, 0][None, :]
    pd = jnp.where(same, pd, -jnp.inf)
    col = jax.lax.broadcasted_iota(jnp.int32, pd.shape, 1)
    for t in range(_K):
        am = jnp.argmax(pd, axis=1).astype(jnp.int32)
        idx_ref[0, t, :] = am
        pd = jnp.where(col == am[:, None], -jnp.inf, pd)
    for t in range(_K, _KPAD):
        idx_ref[0, t, :] = jnp.zeros((_RB,), jnp.int32)


def _knn(xf, tags3):
    b, d, n = xf.shape
    nb = n // _RB
    return pl.pallas_call(
        _topk_body,
        grid=(b, nb),
        in_specs=[
            pl.BlockSpec((1, d, _RB), lambda bi, i: (bi, 0, i)),
            pl.BlockSpec((1, d, n), lambda bi, i: (bi, 0, 0)),
            pl.BlockSpec((1, 1, _RB), lambda bi, i: (bi, 0, i)),
            pl.BlockSpec((1, 1, n), lambda bi, i: (bi, 0, 0)),
        ],
        out_specs=pl.BlockSpec((1, _KPAD, _RB), lambda bi, i: (bi, 0, i)),
        out_shape=jax.ShapeDtypeStruct((b, _KPAD, n), jnp.int32),
    )(xf, xf, tags3, tags3)


def _pad24(xf):
    d = xf.shape[1]
    dp = (-d) % 24
    if dp:
        xf = jnp.pad(xf, ((0, 0), (0, dp), (0, 0)))
    return xf


_NB = 256


def _edge_body(dp, feat_ref, xf_ref, out_ref):
    X = feat_ref[0, 0]
    xc = xf_ref[0]
    XT = jnp.transpose(X)
    out_ref[0, 0, 0:dp, :] = XT - xc
    out_ref[0, 0, dp:2 * dp, :] = xc


def _edge(feat, xfp):
    b, k, n, dp = feat.shape
    nblk = n // _NB
    return pl.pallas_call(
        functools.partial(_edge_body, dp),
        grid=(b, nblk, k),
        in_specs=[
            pl.BlockSpec((1, 1, _NB, dp), lambda bi, i, t: (bi, t, i, 0)),
            pl.BlockSpec((1, dp, _NB), lambda bi, i, t: (bi, 0, i)),
        ],
        out_specs=pl.BlockSpec((1, 1, 2 * dp, _NB),
                               lambda bi, i, t: (bi, t, 0, i)),
        out_shape=jax.ShapeDtypeStruct((b, k, 2 * dp, n), jnp.float32),
    )(feat, xfp)


def _tr(x, node_tag, k):
    bs = x.shape[0]
    n = x.shape[-1]
    xf = x.reshape(bs, -1, n)
    xfp = _pad24(xf)
    dp = xfp.shape[1]
    tags3 = node_tag.astype(jnp.int32).reshape(bs, 1, n)
    idxt = _knn(xfp, tags3)[:, :k, :]
    xt = jnp.transpose(xfp, (0, 2, 1))
    feat = jax.vmap(lambda a, i: a[i])(xt, idxt)
    h = _edge(feat, xfp)
    d = xf.shape[1]
    h = jnp.concatenate([h[:, :, 0:d], h[:, :, dp:dp + d]], axis=2)
    return h.reshape(bs, k, 2 * (d // 3), 3, n)


def _vnbn(x):
    norm = jnp.sqrt(jnp.sum(x * x, axis=2)) + _EPS
    axes = (0,) + tuple(range(2, norm.ndim))
    mean = jnp.mean(norm, axis=axes, keepdims=True)
    var = jnp.var(norm, axis=axes, keepdims=True)
    norm_bn = (norm - mean) / jnp.sqrt(var + 1e-5)
    return x / jnp.expand_dims(norm, 2) * jnp.expand_dims(norm_bn, 2)


def _vnll(x, W, D, ns=0.2):
    p = jnp.einsum('bi...,io->bo...', x, W)
    p = _vnbn(p)
    d = jnp.einsum('bi...,io->bo...', x, D)
    dot = jnp.sum(p * d, axis=2, keepdims=True)
    mask = (dot >= 0).astype(x.dtype)
    dns = jnp.sum(d * d, axis=2, keepdims=True)
    return ns * p + (1.0 - ns) * (mask * p + (1.0 - mask) * (p - (dot / (dns + _EPS)) * d))


def _vnmax(x, D):
    d = jnp.einsum('bki3n,io->bko3n'.replace('3', 'c'), x, D)
    dot = jnp.sum(x * d, axis=3)
    idx = jnp.argmax(dot, axis=1)
    k = x.shape[1]
    sel = (jnp.arange(k, dtype=jnp.int32)[None, :, None, None]
           == idx[:, None, :, :]).astype(x.dtype)
    return jnp.sum(x * sel[:, :, :, None, :], axis=1)


def _vnbn_e(x):
    norm = jnp.sqrt(jnp.sum(x * x, axis=3)) + _EPS
    mean = jnp.mean(norm, axis=(0, 1, 3), keepdims=True)
    var = jnp.var(norm, axis=(0, 1, 3), keepdims=True)
    norm_bn = (norm - mean) / jnp.sqrt(var + 1e-5)
    return x / jnp.expand_dims(norm, 3) * jnp.expand_dims(norm_bn, 3)


def _padw(W, ndp):
    nd = W.shape[0] // 2
    Wp = jnp.zeros((2 * ndp, W.shape[1]), W.dtype)
    Wp = Wp.at[0:nd].set(W[:nd]).at[ndp:ndp + nd].set(W[nd:])
    return Wp


def _vnll_e(x, W, D, ns=0.2):
    p = jnp.einsum('bkicn,io->bkocn', x, W)
    p = _vnbn_e(p)
    d = jnp.einsum('bkicn,io->bkocn', x, D)
    dot = jnp.sum(p * d, axis=3, keepdims=True)
    mask = (dot >= 0).astype(x.dtype)
    dns = jnp.sum(d * d, axis=3, keepdims=True)
    return ns * p + (1.0 - ns) * (mask * p + (1.0 - mask) * (p - (dot / (dns + _EPS)) * d))


def _bn1(h):
    mean = jnp.mean(h, axis=0, keepdims=True)
    var = jnp.var(h, axis=0, keepdims=True)
    return (h - mean) / jnp.sqrt(var + 1e-5)


def kernel(x, node_tag, W1, D1, Dp1, W2, D2, Dp2, W3, D3, Dp3, W4, D4, Dp4,
           W5, D5, Ws1, Ds1, Ws2, Ds2, Wlin, Wl1, Wl2, Wl3, bl1, bl2, bl3):
    bs, _, n = x.shape
    h = _tr(x[:, None, :, :], node_tag, _K)
    h = _vnll_e(h, W1, D1); x1 = _vnmax(h, Dp1)
    h = _tr(x1, node_tag, _K)
    h = _vnll_e(h, W2, D2); x2 = _vnmax(h, Dp2)
    h = _tr(x2, node_tag, _K)
    h = _vnll_e(h, W3, D3); x3 = _vnmax(h, Dp3)
    h = _tr(x3, node_tag, _K)
    h = _vnll_e(h, W4, D4); x4 = _vnmax(h, Dp4)
    xc = jnp.concatenate([x1, x2, x3, x4], axis=1)
    h = _vnll(xc, W5, D5)
    xm = jnp.broadcast_to(jnp.mean(h, axis=-1, keepdims=True), h.shape)
    h = jnp.concatenate([h, xm], axis=1)
    z = _vnll(h, Ws1, Ds1)
    z = _vnll(z, Ws2, Ds2)
    z0 = jnp.einsum('bi...,io->bo...', z, Wlin)
    z0 = jnp.transpose(z0, (0, 2, 1, 3))
    xs = jnp.einsum('bijm,bjkm->bikm', h, z0)
    xf = xs.reshape(bs, -1, n)
    g = jnp.concatenate([jnp.max(xf, axis=-1), jnp.mean(xf, axis=-1)], axis=1)
    g = jax.nn.leaky_relu(_bn1(g @ Wl1 + bl1), 0.2)
    g = jax.nn.leaky_relu(_bn1(g @ Wl2 + bl2), 0.2)
    return g @ Wl3 + bl3


# speedup vs baseline: 1.2524x; 1.2524x over previous
import functools
import jax
import jax.numpy as jnp
import numpy as np
from jax.experimental import pallas as pl
from jax.experimental.pallas import tpu as pltpu

_EPS = 1e-6
_K = 20
_KPAD = 32
_RB = 256


def _topk_body(xfb_ref, xfa_ref, tagb_ref, taga_ref, idx_ref):
    xb = xfb_ref[0]
    xa = xfa_ref[0]
    dot = jax.lax.dot_general(xb, xa, (((0,), (0,)), ((), ())),
                              preferred_element_type=jnp.float32)
    sqb = jnp.sum(xb * xb, axis=0)
    sqa = jnp.sum(xa * xa, axis=0)
    pd = 2.0 * dot - sqb[:, None] - sqa[None, :]
    same = tagb_ref[0, 0][:, None] == taga_ref[0, 0][None, :]
    pd = jnp.where(same, pd, -jnp.inf)
    col = jax.lax.broadcasted_iota(jnp.int32, pd.shape, 1)
    for t in range(_K):
        am = jnp.argmax(pd, axis=1).astype(jnp.int32)
        idx_ref[0, t, :] = am
        pd = jnp.where(col == am[:, None], -jnp.inf, pd)
    for t in range(_K, _KPAD):
        idx_ref[0, t, :] = jnp.zeros((_RB,), jnp.int32)


def _knn(xf, tags3):
    b, d, n = xf.shape
    nb = n // _RB
    return pl.pallas_call(
        _topk_body,
        grid=(b, nb),
        in_specs=[
            pl.BlockSpec((1, d, _RB), lambda bi, i: (bi, 0, i)),
            pl.BlockSpec((1, d, n), lambda bi, i: (bi, 0, 0)),
            pl.BlockSpec((1, 1, _RB), lambda bi, i: (bi, 0, i)),
            pl.BlockSpec((1, 1, n), lambda bi, i: (bi, 0, 0)),
        ],
        out_specs=pl.BlockSpec((1, _KPAD, _RB), lambda bi, i: (bi, 0, i)),
        out_shape=jax.ShapeDtypeStruct((b, _KPAD, n), jnp.int32),
    )(xf, xf, tags3, tags3)


def _pad8(xf):
    d = xf.shape[1]
    dp = (-d) % 8
    if dp:
        xf = jnp.pad(xf, ((0, 0), (0, dp), (0, 0)))
    return xf


def _tr(x, node_tag, k):
    bs = x.shape[0]
    n = x.shape[-1]
    xf = x.reshape(bs, -1, n)
    nd = xf.shape[1] // 3
    tags3 = node_tag.astype(jnp.int32).reshape(bs, 1, n)
    idxt = _knn(_pad8(xf), tags3)[:, :k, :]
    xt = jnp.transpose(xf, (0, 2, 1))
    feat = jax.vmap(lambda a, i: a[i])(xt, idxt)
    feat = jnp.transpose(feat, (0, 1, 3, 2)).reshape(bs, k, nd, 3, n)
    xc = xf.reshape(bs, 1, nd, 3, n)
    out = jnp.concatenate(
        [feat - xc, jnp.broadcast_to(xc, (bs, k, nd, 3, n))], axis=2)
    return out


def _vnbn(x):
    norm = jnp.sqrt(jnp.sum(x * x, axis=2)) + _EPS
    axes = (0,) + tuple(range(2, norm.ndim))
    mean = jnp.mean(norm, axis=axes, keepdims=True)
    var = jnp.var(norm, axis=axes, keepdims=True)
    norm_bn = (norm - mean) / jnp.sqrt(var + 1e-5)
    return x / jnp.expand_dims(norm, 2) * jnp.expand_dims(norm_bn, 2)


def _vnll(x, W, D, ns=0.2):
    p = jnp.einsum('bi...,io->bo...', x, W)
    p = _vnbn(p)
    d = jnp.einsum('bi...,io->bo...', x, D)
    dot = jnp.sum(p * d, axis=2, keepdims=True)
    mask = (dot >= 0).astype(x.dtype)
    dns = jnp.sum(d * d, axis=2, keepdims=True)
    return ns * p + (1.0 - ns) * (mask * p + (1.0 - mask) * (p - (dot / (dns + _EPS)) * d))


def _vnmax(x, D):
    d = jnp.einsum('bki3n,io->bko3n'.replace('3', 'c'), x, D)
    dot = jnp.sum(x * d, axis=3)
    idx = jnp.argmax(dot, axis=1)
    k = x.shape[1]
    sel = (jnp.arange(k, dtype=jnp.int32)[None, :, None, None]
           == idx[:, None, :, :]).astype(x.dtype)
    return jnp.sum(x * sel[:, :, :, None, :], axis=1)


def _vnbn_e(x):
    norm = jnp.sqrt(jnp.sum(x * x, axis=3)) + _EPS
    mean = jnp.mean(norm, axis=(0, 1, 3), keepdims=True)
    var = jnp.var(norm, axis=(0, 1, 3), keepdims=True)
    norm_bn = (norm - mean) / jnp.sqrt(var + 1e-5)
    return x / jnp.expand_dims(norm, 3) * jnp.expand_dims(norm_bn, 3)


def _padw(W, ndp):
    nd = W.shape[0] // 2
    Wp = jnp.zeros((2 * ndp, W.shape[1]), W.dtype)
    Wp = Wp.at[0:nd].set(W[:nd]).at[ndp:ndp + nd].set(W[nd:])
    return Wp


def _vnll_e(x, W, D, ns=0.2):
    p = jnp.einsum('bkicn,io->bkocn', x, W)
    p = _vnbn_e(p)
    d = jnp.einsum('bkicn,io->bkocn', x, D)
    dot = jnp.sum(p * d, axis=3, keepdims=True)
    mask = (dot >= 0).astype(x.dtype)
    dns = jnp.sum(d * d, axis=3, keepdims=True)
    return ns * p + (1.0 - ns) * (mask * p + (1.0 - mask) * (p - (dot / (dns + _EPS)) * d))


def _bn1(h):
    mean = jnp.mean(h, axis=0, keepdims=True)
    var = jnp.var(h, axis=0, keepdims=True)
    return (h - mean) / jnp.sqrt(var + 1e-5)


def kernel(x, node_tag, W1, D1, Dp1, W2, D2, Dp2, W3, D3, Dp3, W4, D4, Dp4,
           W5, D5, Ws1, Ds1, Ws2, Ds2, Wlin, Wl1, Wl2, Wl3, bl1, bl2, bl3):
    bs, _, n = x.shape
    h = _tr(x[:, None, :, :], node_tag, _K)
    h = _vnll_e(h, W1, D1); x1 = _vnmax(h, Dp1)
    h = _tr(x1, node_tag, _K)
    h = _vnll_e(h, W2, D2); x2 = _vnmax(h, Dp2)
    h = _tr(x2, node_tag, _K)
    h = _vnll_e(h, W3, D3); x3 = _vnmax(h, Dp3)
    h = _tr(x3, node_tag, _K)
    h = _vnll_e(h, W4, D4); x4 = _vnmax(h, Dp4)
    xc = jnp.concatenate([x1, x2, x3, x4], axis=1)
    h = _vnll(xc, W5, D5)
    xm = jnp.broadcast_to(jnp.mean(h, axis=-1, keepdims=True), h.shape)
    h = jnp.concatenate([h, xm], axis=1)
    z = _vnll(h, Ws1, Ds1)
    z = _vnll(z, Ws2, Ds2)
    z0 = jnp.einsum('bi...,io->bo...', z, Wlin)
    z0 = jnp.transpose(z0, (0, 2, 1, 3))
    xs = jnp.einsum('bijm,bjkm->bikm', h, z0)
    xf = xs.reshape(bs, -1, n)
    g = jnp.concatenate([jnp.max(xf, axis=-1), jnp.mean(xf, axis=-1)], axis=1)
    g = jax.nn.leaky_relu(_bn1(g @ Wl1 + bl1), 0.2)
    g = jax.nn.leaky_relu(_bn1(g @ Wl2 + bl2), 0.2)
    return g @ Wl3 + bl3


# speedup vs baseline: 1.2528x; 1.0003x over previous
import functools
import jax
import jax.numpy as jnp
import numpy as np
from jax.experimental import pallas as pl
from jax.experimental.pallas import tpu as pltpu

_EPS = 1e-6
_K = 20
_KPAD = 32
_RB = 256


def _topk_body(xfb_ref, xfa_ref, tagb_ref, taga_ref, idx_ref):
    xb = xfb_ref[0]
    xa = xfa_ref[0]
    dot = jax.lax.dot_general(xb, xa, (((0,), (0,)), ((), ())),
                              preferred_element_type=jnp.float32)
    sqb = jnp.sum(xb * xb, axis=0)
    sqa = jnp.sum(xa * xa, axis=0)
    pd = 2.0 * dot - sqb[:, None] - sqa[None, :]
    same = tagb_ref[0, 0][:, None] == taga_ref[0, 0][None, :]
    pd = jnp.where(same, pd, -jnp.inf)
    col = jax.lax.broadcasted_iota(jnp.int32, pd.shape, 1)
    for t in range(_K):
        am = jnp.argmax(pd, axis=1).astype(jnp.int32)
        idx_ref[0, t, :] = am
        pd = jnp.where(col == am[:, None], -jnp.inf, pd)
    for t in range(_K, _KPAD):
        idx_ref[0, t, :] = jnp.zeros((_RB,), jnp.int32)


def _knn(xf, tags3):
    b, d, n = xf.shape
    nb = n // _RB
    return pl.pallas_call(
        _topk_body,
        grid=(b, nb),
        in_specs=[
            pl.BlockSpec((1, d, _RB), lambda bi, i: (bi, 0, i)),
            pl.BlockSpec((1, d, n), lambda bi, i: (bi, 0, 0)),
            pl.BlockSpec((1, 1, _RB), lambda bi, i: (bi, 0, i)),
            pl.BlockSpec((1, 1, n), lambda bi, i: (bi, 0, 0)),
        ],
        out_specs=pl.BlockSpec((1, _KPAD, _RB), lambda bi, i: (bi, 0, i)),
        out_shape=jax.ShapeDtypeStruct((b, _KPAD, n), jnp.int32),
    )(xf, xf, tags3, tags3)


def _pad8(xf):
    d = xf.shape[1]
    dp = (-d) % 8
    if dp:
        xf = jnp.pad(xf, ((0, 0), (0, dp), (0, 0)))
    return xf


def _tr(x, node_tag, k):
    bs = x.shape[0]
    n = x.shape[-1]
    xf = x.reshape(bs, -1, n)
    nd = xf.shape[1] // 3
    tags3 = node_tag.astype(jnp.int32).reshape(bs, 1, n)
    idxt = _knn(_pad8(xf), tags3)[:, :k, :]
    xt = jnp.transpose(xf, (0, 2, 1))
    feat = jax.vmap(lambda a, i: a[i])(xt, idxt)
    feat = jnp.transpose(feat, (0, 1, 3, 2)).reshape(bs, k, nd, 3, n)
    xc = xf.reshape(bs, 1, nd, 3, n)
    out = jnp.concatenate(
        [feat - xc, jnp.broadcast_to(xc, (bs, k, nd, 3, n))], axis=2)
    return out


def _vnbn(x):
    norm = jnp.sqrt(jnp.sum(x * x, axis=2)) + _EPS
    axes = (0,) + tuple(range(2, norm.ndim))
    mean = jnp.mean(norm, axis=axes, keepdims=True)
    var = jnp.var(norm, axis=axes, keepdims=True)
    norm_bn = (norm - mean) / jnp.sqrt(var + 1e-5)
    return x / jnp.expand_dims(norm, 2) * jnp.expand_dims(norm_bn, 2)


def _vnll(x, W, D, ns=0.2):
    p = jnp.einsum('bi...,io->bo...', x, W)
    p = _vnbn(p)
    d = jnp.einsum('bi...,io->bo...', x, D)
    dot = jnp.sum(p * d, axis=2, keepdims=True)
    mask = (dot >= 0).astype(x.dtype)
    dns = jnp.sum(d * d, axis=2, keepdims=True)
    return ns * p + (1.0 - ns) * (mask * p + (1.0 - mask) * (p - (dot / (dns + _EPS)) * d))


def _vnmax(x, D):
    d = jnp.einsum('bkicn,io->bkocn', x, D)
    dot = jnp.sum(x * d, axis=3)
    idx = jnp.argmax(dot, axis=1)
    k = x.shape[1]
    sel = (jnp.arange(k, dtype=jnp.int32)[None, :, None, None]
           == idx[:, None, :, :]).astype(x.dtype)
    return jnp.sum(x * sel[:, :, :, None, :], axis=1)


def _vnbn_e(x):
    norm = jnp.sqrt(jnp.sum(x * x, axis=3)) + _EPS
    mean = jnp.mean(norm, axis=(0, 1, 3), keepdims=True)
    var = jnp.var(norm, axis=(0, 1, 3), keepdims=True)
    norm_bn = (norm - mean) / jnp.sqrt(var + 1e-5)
    return x / jnp.expand_dims(norm, 3) * jnp.expand_dims(norm_bn, 3)


def _vnll_e(x, W, D, ns=0.2):
    p = jnp.einsum('bkicn,io->bkocn', x, W)
    p = _vnbn_e(p)
    d = jnp.einsum('bkicn,io->bkocn', x, D)
    dot = jnp.sum(p * d, axis=3, keepdims=True)
    mask = (dot >= 0).astype(x.dtype)
    dns = jnp.sum(d * d, axis=3, keepdims=True)
    return ns * p + (1.0 - ns) * (mask * p + (1.0 - mask) * (p - (dot / (dns + _EPS)) * d))


def _bn1(h):
    mean = jnp.mean(h, axis=0, keepdims=True)
    var = jnp.var(h, axis=0, keepdims=True)
    return (h - mean) / jnp.sqrt(var + 1e-5)


def kernel(x, node_tag, W1, D1, Dp1, W2, D2, Dp2, W3, D3, Dp3, W4, D4, Dp4,
           W5, D5, Ws1, Ds1, Ws2, Ds2, Wlin, Wl1, Wl2, Wl3, bl1, bl2, bl3):
    bs, _, n = x.shape
    h = _tr(x[:, None, :, :], node_tag, _K)
    h = _vnll_e(h, W1, D1); x1 = _vnmax(h, Dp1)
    h = _tr(x1, node_tag, _K)
    h = _vnll_e(h, W2, D2); x2 = _vnmax(h, Dp2)
    h = _tr(x2, node_tag, _K)
    h = _vnll_e(h, W3, D3); x3 = _vnmax(h, Dp3)
    h = _tr(x3, node_tag, _K)
    h = _vnll_e(h, W4, D4); x4 = _vnmax(h, Dp4)
    xc = jnp.concatenate([x1, x2, x3, x4], axis=1)
    h = _vnll(xc, W5, D5)
    xm = jnp.broadcast_to(jnp.mean(h, axis=-1, keepdims=True), h.shape)
    h = jnp.concatenate([h, xm], axis=1)
    z = _vnll(h, Ws1, Ds1)
    z = _vnll(z, Ws2, Ds2)
    z0 = jnp.einsum('bi...,io->bo...', z, Wlin)
    z0 = jnp.transpose(z0, (0, 2, 1, 3))
    xs = jnp.einsum('bijm,bjkm->bikm', h, z0)
    xf = xs.reshape(bs, -1, n)
    g = jnp.concatenate([jnp.max(xf, axis=-1), jnp.mean(xf, axis=-1)], axis=1)
    g = jax.nn.leaky_relu(_bn1(g @ Wl1 + bl1), 0.2)
    g = jax.nn.leaky_relu(_bn1(g @ Wl2 + bl2), 0.2)
    return g @ Wl3 + bl3
